# Initial kernel scaffold; baseline (speedup 1.0000x reference)
import jax
import jax.numpy as jnp
from jax.experimental import pallas as pl


def kernel(image_shape, boxes, scores, fpn0, fpn1, fpn2, fpn3, fpn4):
    raise NotImplementedError("write your pallas kernel here")


# speedup vs baseline: 22.1645x; 22.1645x over previous
import functools

import jax
import jax.numpy as jnp
from jax import lax
from jax.experimental import pallas as pl
from jax.experimental.pallas import tpu as pltpu
from jax.experimental.pallas import tpu_sc as plsc

CROP = 14
PTS = CROP * CROP
NC, NS, LANES = 2, 16, 16
NW = NC * NS
KCH = 128
BN = 256

FPN_SIZES = (64, 32, 16, 8, 4)
LVL_OFF = (0, 4096, 5120, 5376, 5440)
NROWS = 5456


def _lane_bcast(vec, pv):
    dnums = lax.GatherDimensionNumbers(
        offset_dims=(), collapsed_slice_dims=(0,), start_index_map=(0,))
    return lax.gather(vec, pv[:, None], dnums, (1,),
                      mode=lax.GatherScatterMode.PROMISE_IN_BOUNDS)


def _level_coords(boxes_b, img_h, img_w):
    x1, y1 = boxes_b[:, 0], boxes_b[:, 1]
    x2, y2 = boxes_b[:, 2], boxes_b[:, 3]
    w = x2 - x1
    h = y2 - y1
    size = jnp.sqrt(w * h)
    log2 = jnp.log(size / 224.0 + 1e-7) / jnp.log(2.0)
    levels = jnp.clip(jnp.floor(1.0 + log2), 0.0, 4.0)
    ty = jnp.linspace(0.0, 1.0, CROP)
    tx = jnp.linspace(0.0, 1.0, CROP)
    iy = jnp.zeros((boxes_b.shape[0], CROP), jnp.float32)
    ix = jnp.zeros((boxes_b.shape[0], CROP), jnp.float32)
    for i, Hs in enumerate(FPN_SIZES):
        Hf = float(Hs)
        Wf = float(Hs)
        y1n = y1 / img_h * Hf / (Hf - 1.0)
        x1n = x1 / img_w * Wf / (Wf - 1.0)
        y2n = (y2 / img_h * Hf - 1.0) / (Hf - 1.0)
        x2n = (x2 / img_w * Wf - 1.0) / (Wf - 1.0)
        in_y = y1n[:, None] * (Hf - 1.0) + ty[None, :] * (
            (y2n - y1n)[:, None] * (Hf - 1.0))
        in_x = x1n[:, None] * (Wf - 1.0) + tx[None, :] * (
            (x2n - x1n)[:, None] * (Wf - 1.0))
        sel = (levels == float(i))[:, None]
        iy = jnp.where(sel, in_y, iy)
        ix = jnp.where(sel, in_x, ix)
    return levels, iy, ix


def _tc_prep_body(iy_ref, ix_ref, lvl_ref, i0, i1, i2, i3, w0, w1, w2, w3):
    f32 = jnp.float32
    lvl = lvl_ref[:, 0:1]

    Hf = jnp.full_like(lvl, float(FPN_SIZES[-1]))
    off = jnp.full_like(lvl, float(LVL_OFF[-1]))
    for li in range(len(FPN_SIZES) - 2, -1, -1):
        sel = lvl <= (li + 0.5)
        Hf = jnp.where(sel, float(FPN_SIZES[li]), Hf)
        off = jnp.where(sel, float(LVL_OFF[li]), off)
    Hm1 = Hf - 1.0

    in_y = iy_ref[...]
    in_x = ix_ref[...]
    valid = (in_y >= 0.0) & (in_y <= Hm1) & (in_x >= 0.0) & (in_x <= Hm1)
    y0f = jnp.floor(in_y)
    x0f = jnp.floor(in_x)
    fy = in_y - y0f
    fx = in_x - x0f
    y0i = jnp.clip(y0f, 0.0, Hm1).astype(jnp.int32)
    y1i = jnp.clip(y0f + 1.0, 0.0, Hm1).astype(jnp.int32)
    x0i = jnp.clip(x0f, 0.0, Hm1).astype(jnp.int32)
    x1i = jnp.clip(x0f + 1.0, 0.0, Hm1).astype(jnp.int32)

    m = valid.astype(f32)
    omfy = 1.0 - fy
    omfx = 1.0 - fx
    w0[...] = omfy * omfx * m
    w1[...] = omfy * fx * m
    w2[...] = fy * omfx * m
    w3[...] = fy * fx * m

    offi = off.astype(jnp.int32)
    Wi = Hf.astype(jnp.int32)
    row0 = offi + y0i * Wi
    row1 = offi + y1i * Wi
    i0[...] = row0 + x0i
    i1[...] = row0 + x1i
    i2[...] = row1 + x0i
    i3[...] = row1 + x1i


def _tc_prep(iy_e, ix_e, lvl2, npad):
    grid = npad // BN
    blk_c = pl.BlockSpec((BN, PTS), lambda i: (i, 0))
    blk_l = pl.BlockSpec((BN, 1), lambda i: (i, 0))
    oshape = [jax.ShapeDtypeStruct((npad, PTS), jnp.int32)] * 4 + [
        jax.ShapeDtypeStruct((npad, PTS), jnp.float32)
    ] * 4
    return pl.pallas_call(
        _tc_prep_body,
        grid=(grid,),
        in_specs=[blk_c, blk_c, blk_l],
        out_specs=[blk_c] * 8,
        out_shape=oshape,
    )(iy_e, ix_e, lvl2)


def _make_sc_kernel(p_pad, C, CP):
    per_w = p_pad // NW
    n_chunks = per_w // KCH
    cbl = C // LANES
    mesh = plsc.VectorSubcoreMesh(core_axis_name="c", subcore_axis_name="s")

    @functools.partial(
        pl.kernel,
        mesh=mesh,
        out_type=jax.ShapeDtypeStruct((p_pad, C), jnp.float32),
        scratch_types=[
            pltpu.VMEM((KCH,), jnp.int32),
            pltpu.VMEM((KCH,), jnp.int32),
            pltpu.VMEM((KCH,), jnp.int32),
            pltpu.VMEM((KCH,), jnp.int32),
            pltpu.VMEM((KCH,), jnp.float32),
            pltpu.VMEM((KCH,), jnp.float32),
            pltpu.VMEM((KCH,), jnp.float32),
            pltpu.VMEM((KCH,), jnp.float32),
            pltpu.VMEM((KCH, CP), jnp.float32),
            pltpu.VMEM((KCH, CP), jnp.float32),
            pltpu.VMEM((KCH, CP), jnp.float32),
            pltpu.VMEM((KCH, CP), jnp.float32),
            pltpu.VMEM((KCH, C), jnp.float32),
            pltpu.SemaphoreType.DMA,
        ],
    )
    def sc_kernel(F, i0, i1, i2, i3, w0, w1, w2, w3, out,
                  iv0, iv1, iv2, iv3, wv0, wv1, wv2, wv3,
                  t0, t1, t2, t3, ov, sem):
        wid = lax.axis_index("s") * NC + lax.axis_index("c")
        base = wid * per_w

        def chunk_body(ci, carry):
            off = base + ci * KCH
            pltpu.sync_copy(i0.at[pl.ds(off, KCH)], iv0)
            pltpu.sync_copy(i1.at[pl.ds(off, KCH)], iv1)
            pltpu.sync_copy(i2.at[pl.ds(off, KCH)], iv2)
            pltpu.sync_copy(i3.at[pl.ds(off, KCH)], iv3)
            pltpu.sync_copy(w0.at[pl.ds(off, KCH)], wv0)
            pltpu.sync_copy(w1.at[pl.ds(off, KCH)], wv1)
            pltpu.sync_copy(w2.at[pl.ds(off, KCH)], wv2)
            pltpu.sync_copy(w3.at[pl.ds(off, KCH)], wv3)
            cp0 = pltpu.async_copy(F.at[iv0], t0, sem)
            cp1 = pltpu.async_copy(F.at[iv1], t1, sem)
            cp2 = pltpu.async_copy(F.at[iv2], t2, sem)
            cp3 = pltpu.async_copy(F.at[iv3], t3, sem)
            cp0.wait()
            cp1.wait()
            cp2.wait()
            cp3.wait()

            def group_body(g, carry2):
                gsl = pl.ds(g * LANES, LANES)
                g0 = wv0[gsl]
                g1 = wv1[gsl]
                g2 = wv2[gsl]
                g3 = wv3[gsl]
                for j in range(LANES):
                    p = g * LANES + j
                    pv = jnp.full((LANES,), j, jnp.int32)
                    a0 = _lane_bcast(g0, pv)
                    a1 = _lane_bcast(g1, pv)
                    a2 = _lane_bcast(g2, pv)
                    a3 = _lane_bcast(g3, pv)
                    for cb in range(cbl):
                        sl = pl.ds(cb * LANES, LANES)
                        acc = a0 * t0[p, sl]
                        acc = acc + a1 * t1[p, sl]
                        acc = acc + a2 * t2[p, sl]
                        acc = acc + a3 * t3[p, sl]
                        ov[p, sl] = acc
                return carry2

            lax.fori_loop(0, KCH // LANES, group_body, 0, unroll=False)
            pltpu.sync_copy(ov, out.at[pl.ds(off, KCH)])
            return carry

        lax.fori_loop(0, n_chunks, chunk_body, 0, unroll=False)

    return sc_kernel


def kernel(image_shape, boxes, scores, fpn0, fpn1, fpn2, fpn3, fpn4):
    del scores
    fpns = (fpn0, fpn1, fpn2, fpn3, fpn4)
    B, N = boxes.shape[0], boxes.shape[1]
    C = fpn0.shape[-1]
    img = image_shape.astype(jnp.float32)
    img_h, img_w = img[1], img[2]

    npad = BN
    while npad < N or (npad * PTS) % (NW * KCH) != 0:
        npad += BN
    p_pad = npad * PTS

    CP = ((C + 127) // 128) * 128
    sc_fn = _make_sc_kernel(p_pad, C, CP)

    outs = []
    for b in range(B):
        F = jnp.concatenate([f[b].reshape(-1, C) for f in fpns], axis=0)
        F = jnp.pad(F, ((0, 0), (0, CP - C)))
        levels, iy, ix = _level_coords(boxes[b], img_h, img_w)
        iy_e = jnp.pad(jnp.repeat(iy, CROP, axis=1),
                       ((0, npad - N), (0, 0)), constant_values=-1.0)
        ix_e = jnp.pad(jnp.tile(ix, (1, CROP)),
                       ((0, npad - N), (0, 0)), constant_values=-1.0)
        lvl2 = jnp.pad(levels[:, None], ((0, npad - N), (0, 0)))
        prep = _tc_prep(iy_e, ix_e, lvl2, npad)
        flat = [a.reshape(-1) for a in prep]
        out_flat = sc_fn(F, *flat)
        outs.append(out_flat[: N * PTS].reshape(N, CROP, CROP, C))
    return jnp.stack(outs, axis=0)


# speedup vs baseline: 25.1179x; 1.1332x over previous
import functools

import jax
import jax.numpy as jnp
from jax import lax
from jax.experimental import pallas as pl
from jax.experimental.pallas import tpu as pltpu
from jax.experimental.pallas import tpu_sc as plsc

CROP = 14
PTS = CROP * CROP
NC, NS, LANES = 2, 16, 16
NW = NC * NS
KCH = 64
BN = 256

FPN_SIZES = (64, 32, 16, 8, 4)
LVL_OFF = (0, 4096, 5120, 5376, 5440)
NROWS = 5456


def _lane_bcast(vec, pv):
    dnums = lax.GatherDimensionNumbers(
        offset_dims=(), collapsed_slice_dims=(0,), start_index_map=(0,))
    return lax.gather(vec, pv[:, None], dnums, (1,),
                      mode=lax.GatherScatterMode.PROMISE_IN_BOUNDS)


def _level_coords(boxes_b, img_h, img_w):
    x1, y1 = boxes_b[:, 0], boxes_b[:, 1]
    x2, y2 = boxes_b[:, 2], boxes_b[:, 3]
    w = x2 - x1
    h = y2 - y1
    size = jnp.sqrt(w * h)
    log2 = jnp.log(size / 224.0 + 1e-7) / jnp.log(2.0)
    levels = jnp.clip(jnp.floor(1.0 + log2), 0.0, 4.0)
    ty = jnp.linspace(0.0, 1.0, CROP)
    tx = jnp.linspace(0.0, 1.0, CROP)
    iy = jnp.zeros((boxes_b.shape[0], CROP), jnp.float32)
    ix = jnp.zeros((boxes_b.shape[0], CROP), jnp.float32)
    for i, Hs in enumerate(FPN_SIZES):
        Hf = float(Hs)
        Wf = float(Hs)
        y1n = y1 / img_h * Hf / (Hf - 1.0)
        x1n = x1 / img_w * Wf / (Wf - 1.0)
        y2n = (y2 / img_h * Hf - 1.0) / (Hf - 1.0)
        x2n = (x2 / img_w * Wf - 1.0) / (Wf - 1.0)
        in_y = y1n[:, None] * (Hf - 1.0) + ty[None, :] * (
            (y2n - y1n)[:, None] * (Hf - 1.0))
        in_x = x1n[:, None] * (Wf - 1.0) + tx[None, :] * (
            (x2n - x1n)[:, None] * (Wf - 1.0))
        sel = (levels == float(i))[:, None]
        iy = jnp.where(sel, in_y, iy)
        ix = jnp.where(sel, in_x, ix)
    return levels, iy, ix


def _tc_prep_body(iy_ref, ix_ref, lvl_ref, o_ref):
    f32 = jnp.float32
    lvl = lvl_ref[:, 0:1]

    Hf = jnp.full_like(lvl, float(FPN_SIZES[-1]))
    off = jnp.full_like(lvl, float(LVL_OFF[-1]))
    for li in range(len(FPN_SIZES) - 2, -1, -1):
        sel = lvl <= (li + 0.5)
        Hf = jnp.where(sel, float(FPN_SIZES[li]), Hf)
        off = jnp.where(sel, float(LVL_OFF[li]), off)
    Hm1 = Hf - 1.0

    in_y = iy_ref[...]
    in_x = ix_ref[...]
    valid = (in_y >= 0.0) & (in_y <= Hm1) & (in_x >= 0.0) & (in_x <= Hm1)
    y0f = jnp.floor(in_y)
    x0f = jnp.floor(in_x)
    fy = in_y - y0f
    fx = in_x - x0f
    y0i = jnp.clip(y0f, 0.0, Hm1).astype(jnp.int32)
    y1i = jnp.clip(y0f + 1.0, 0.0, Hm1).astype(jnp.int32)
    x0i = jnp.clip(x0f, 0.0, Hm1).astype(jnp.int32)
    x1i = jnp.clip(x0f + 1.0, 0.0, Hm1).astype(jnp.int32)

    m = valid.astype(f32)
    omfy = 1.0 - fy
    omfx = 1.0 - fx
    bc = lax.bitcast_convert_type
    offi = off.astype(jnp.int32)
    Wi = Hf.astype(jnp.int32)
    row0 = offi + y0i * Wi
    row1 = offi + y1i * Wi
    o_ref[0] = row0 + x0i
    o_ref[1] = row0 + x1i
    o_ref[2] = row1 + x0i
    o_ref[3] = row1 + x1i
    o_ref[4] = bc(omfy * omfx * m, jnp.int32)
    o_ref[5] = bc(omfy * fx * m, jnp.int32)
    o_ref[6] = bc(fy * omfx * m, jnp.int32)
    o_ref[7] = bc(fy * fx * m, jnp.int32)


def _tc_prep(iy_e, ix_e, lvl2, npad):
    grid = npad // BN
    blk_c = pl.BlockSpec((BN, PTS), lambda i: (i, 0))
    blk_l = pl.BlockSpec((BN, 1), lambda i: (i, 0))
    blk_o = pl.BlockSpec((8, BN, PTS), lambda i: (0, i, 0))
    return pl.pallas_call(
        _tc_prep_body,
        grid=(grid,),
        in_specs=[blk_c, blk_c, blk_l],
        out_specs=blk_o,
        out_shape=jax.ShapeDtypeStruct((8, npad, PTS), jnp.int32),
    )(iy_e, ix_e, lvl2)


def _make_sc_kernel(p_real, p_pad, C, CP):
    per_w = p_pad // NW
    mesh = plsc.VectorSubcoreMesh(core_axis_name="c", subcore_axis_name="s")

    @functools.partial(
        pl.kernel,
        mesh=mesh,
        out_type=jax.ShapeDtypeStruct((p_pad, C), jnp.float32),
        scratch_types=[
            pltpu.VMEM((per_w,), jnp.float32) for _ in range(4)
        ] + [pltpu.VMEM((KCH,), jnp.int32) for _ in range(8)
        ] + [pltpu.VMEM((KCH, CP), jnp.float32) for _ in range(8)] + [
            pltpu.VMEM((KCH, C), jnp.float32),
            pltpu.VMEM((KCH, C), jnp.float32),
        ] + [pltpu.SemaphoreType.DMA for _ in range(6)],
    )
    def sc_kernel(F, i0, i1, i2, i3, w0, w1, w2, w3, out,
                  sw0, sw1, sw2, sw3,
                  x00, x01, x02, x03, x10, x11, x12, x13,
                  t00, t01, t02, t03, t10, t11, t12, t13,
                  ov0, ov1, sg0, sg1, so0, so1, ss0, ss1):
        wid = lax.axis_index("s") * NC + lax.axis_index("c")
        base = wid * per_w
        nch = per_w // KCH
        iref = (i0, i1, i2, i3)
        wref = (w0, w1, w2, w3)
        stgw = (sw0, sw1, sw2, sw3)
        sidx = ((x00, x01, x02, x03), (x10, x11, x12, x13))
        taps = ((t00, t01, t02, t03), (t10, t11, t12, t13))
        ov = (ov0, ov1)
        gsem = (sg0, sg1)
        osem = (so0, so1)
        ssem = (ss0, ss1)

        for t in range(4):
            pltpu.sync_copy(wref[t].at[pl.ds(base, per_w)], stgw[t])

        def stage(b, ci):
            return [
                pltpu.make_async_copy(
                    iref[t].at[pl.ds(base + ci * KCH, KCH)],
                    sidx[b][t], ssem[b])
                for t in range(4)
            ]

        def gathers(b, ci):
            del ci
            return [
                pltpu.make_async_copy(
                    F.at[sidx[b][t].at[pl.ds(0, KCH)]],
                    taps[b][t], gsem[b])
                for t in range(4)
            ]

        def store(b, ci):
            return pltpu.make_async_copy(
                ov[b], out.at[pl.ds(base + ci * KCH, KCH)], osem[b])

        def combine(b, ci):
            tb = taps[b]
            ovb = ov[b]
            wbase = ci * KCH

            def group_body(g, carry):
                gsl = pl.ds(wbase + g * LANES, LANES)
                gw = [stgw[t][gsl] for t in range(4)]
                for j in range(LANES):
                    p = g * LANES + j
                    pv = jnp.full((LANES,), j, jnp.int32)
                    a = [_lane_bcast(gw[t], pv) for t in range(4)]
                    for cb in range(C // LANES):
                        s16 = pl.ds(cb * LANES, LANES)
                        acc = None
                        for t in range(4):
                            u = tb[t][p, s16]
                            acc = a[t] * u if acc is None else acc + a[t] * u
                        ovb[p, s16] = acc
                return carry

            lax.fori_loop(0, KCH // LANES, group_body, 0, unroll=False)

        for cp in stage(0, 0):
            cp.start()
        for cp in stage(0, 0):
            cp.wait()
        for cp in gathers(0, 0):
            cp.start()
        for cp in stage(1, 1):
            cp.start()

        def half(ci, b):
            nb = 1 - b
            for cp in gathers(b, ci):
                cp.wait()

            @pl.when(ci + 1 < nch)
            def _():
                for cp in stage(nb, ci + 1):
                    cp.wait()
                for cp in gathers(nb, ci + 1):
                    cp.start()

            @pl.when(ci + 2 < nch)
            def _():
                for cp in stage(b, ci + 2):
                    cp.start()

            @pl.when(ci >= 2)
            def _():
                store(b, ci - 2).wait()

            combine(b, ci)
            store(b, ci).start()

        def body(ci2, carry):
            half(ci2 * 2, 0)
            half(ci2 * 2 + 1, 1)
            return carry

        lax.fori_loop(0, nch // 2, body, 0, unroll=False)
        store(0, nch - 2).wait()
        store(1, nch - 1).wait()

    return sc_kernel


def kernel(image_shape, boxes, scores, fpn0, fpn1, fpn2, fpn3, fpn4):
    del scores
    fpns = (fpn0, fpn1, fpn2, fpn3, fpn4)
    B, N = boxes.shape[0], boxes.shape[1]
    C = fpn0.shape[-1]
    img = image_shape.astype(jnp.float32)
    img_h, img_w = img[1], img[2]

    npad = BN
    while npad < N or (npad * PTS) % (NW * KCH) != 0:
        npad += BN
    p_pad = npad * PTS

    CP = ((C + 127) // 128) * 128
    p_real = N * PTS
    sc_fn = _make_sc_kernel(p_real, p_pad, C, CP)

    outs = []
    for b in range(B):
        F = jnp.concatenate([f[b].reshape(-1, C) for f in fpns], axis=0)
        F = jnp.pad(F, ((0, 0), (0, CP - C)))
        levels, iy, ix = _level_coords(boxes[b], img_h, img_w)
        iy_e = jnp.pad(jnp.repeat(iy, CROP, axis=1),
                       ((0, npad - N), (0, 0)), constant_values=-1.0)
        ix_e = jnp.pad(jnp.tile(ix, (1, CROP)),
                       ((0, npad - N), (0, 0)), constant_values=-1.0)
        lvl2 = jnp.pad(levels[:, None], ((0, npad - N), (0, 0)))
        prep = _tc_prep(iy_e, ix_e, lvl2, npad)
        idxw = prep.reshape(8, npad * PTS)
        idxs = [idxw[t] for t in range(4)]
        wts = [lax.bitcast_convert_type(idxw[4 + t], jnp.float32)
               for t in range(4)]
        out_flat = sc_fn(F, *idxs, *wts)
        outs.append(out_flat[:N * PTS].reshape(N, CROP, CROP, C))
    return jnp.stack(outs, axis=0)


# speedup vs baseline: 26.1041x; 1.0393x over previous
import functools

import jax
import jax.numpy as jnp
from jax import lax
from jax.experimental import pallas as pl
from jax.experimental.pallas import tpu as pltpu
from jax.experimental.pallas import tpu_sc as plsc

CROP = 14
PTS = CROP * CROP
NC, NS, LANES = 2, 16, 16
NW = NC * NS
KCH = 64
BN = 256

FPN_SIZES = (64, 32, 16, 8, 4)
LVL_OFF = (0, 4096, 5120, 5376, 5440)
NROWS = 5456


def _lane_bcast(vec, pv):
    dnums = lax.GatherDimensionNumbers(
        offset_dims=(), collapsed_slice_dims=(0,), start_index_map=(0,))
    return lax.gather(vec, pv[:, None], dnums, (1,),
                      mode=lax.GatherScatterMode.PROMISE_IN_BOUNDS)


def _level_coords(boxes_b, img_h, img_w):
    x1, y1 = boxes_b[:, 0], boxes_b[:, 1]
    x2, y2 = boxes_b[:, 2], boxes_b[:, 3]
    w = x2 - x1
    h = y2 - y1
    size = jnp.sqrt(w * h)
    log2 = jnp.log(size / 224.0 + 1e-7) / jnp.log(2.0)
    levels = jnp.clip(jnp.floor(1.0 + log2), 0.0, 4.0)
    ty = jnp.linspace(0.0, 1.0, CROP)
    tx = jnp.linspace(0.0, 1.0, CROP)
    iy = jnp.zeros((boxes_b.shape[0], CROP), jnp.float32)
    ix = jnp.zeros((boxes_b.shape[0], CROP), jnp.float32)
    for i, Hs in enumerate(FPN_SIZES):
        Hf = float(Hs)
        Wf = float(Hs)
        y1n = y1 / img_h * Hf / (Hf - 1.0)
        x1n = x1 / img_w * Wf / (Wf - 1.0)
        y2n = (y2 / img_h * Hf - 1.0) / (Hf - 1.0)
        x2n = (x2 / img_w * Wf - 1.0) / (Wf - 1.0)
        in_y = y1n[:, None] * (Hf - 1.0) + ty[None, :] * (
            (y2n - y1n)[:, None] * (Hf - 1.0))
        in_x = x1n[:, None] * (Wf - 1.0) + tx[None, :] * (
            (x2n - x1n)[:, None] * (Wf - 1.0))
        sel = (levels == float(i))[:, None]
        iy = jnp.where(sel, in_y, iy)
        ix = jnp.where(sel, in_x, ix)
    return levels, iy, ix


def _tc_prep_body(iy_ref, ix_ref, lvl_ref,
                  oi0, oi1, oi2, oi3, ow0, ow1, ow2, ow3):
    f32 = jnp.float32
    lvl = lvl_ref[:, 0:1]

    Hf = jnp.full_like(lvl, float(FPN_SIZES[-1]))
    off = jnp.full_like(lvl, float(LVL_OFF[-1]))
    for li in range(len(FPN_SIZES) - 2, -1, -1):
        sel = lvl <= (li + 0.5)
        Hf = jnp.where(sel, float(FPN_SIZES[li]), Hf)
        off = jnp.where(sel, float(LVL_OFF[li]), off)
    Hm1 = Hf - 1.0

    in_y = iy_ref[...]
    in_x = ix_ref[...]
    valid = (in_y >= 0.0) & (in_y <= Hm1) & (in_x >= 0.0) & (in_x <= Hm1)
    y0f = jnp.floor(in_y)
    x0f = jnp.floor(in_x)
    fy = in_y - y0f
    fx = in_x - x0f
    y0i = jnp.clip(y0f, 0.0, Hm1).astype(jnp.int32)
    y1i = jnp.clip(y0f + 1.0, 0.0, Hm1).astype(jnp.int32)
    x0i = jnp.clip(x0f, 0.0, Hm1).astype(jnp.int32)
    x1i = jnp.clip(x0f + 1.0, 0.0, Hm1).astype(jnp.int32)

    m = valid.astype(f32)
    omfy = 1.0 - fy
    omfx = 1.0 - fx
    offi = off.astype(jnp.int32)
    Wi = Hf.astype(jnp.int32)
    row0 = offi + y0i * Wi
    row1 = offi + y1i * Wi
    oi0[...] = row0 + x0i
    oi1[...] = row0 + x1i
    oi2[...] = row1 + x0i
    oi3[...] = row1 + x1i
    ow0[...] = omfy * omfx * m
    ow1[...] = omfy * fx * m
    ow2[...] = fy * omfx * m
    ow3[...] = fy * fx * m


def _tc_prep(iy_e, ix_e, lvl2, npad):
    grid = npad // BN
    blk_c = pl.BlockSpec((BN, PTS), lambda i: (i, 0))
    blk_l = pl.BlockSpec((BN, 1), lambda i: (i, 0))
    return pl.pallas_call(
        _tc_prep_body,
        grid=(grid,),
        in_specs=[blk_c, blk_c, blk_l],
        out_specs=[blk_c] * 8,
        out_shape=[jax.ShapeDtypeStruct((npad, PTS), jnp.int32)] * 4
        + [jax.ShapeDtypeStruct((npad, PTS), jnp.float32)] * 4,
    )(iy_e, ix_e, lvl2)


def _make_sc_kernel(p_real, p_pad, C, CP):
    per_w = p_pad // NW
    mesh = plsc.VectorSubcoreMesh(core_axis_name="c", subcore_axis_name="s")

    @functools.partial(
        pl.kernel,
        mesh=mesh,
        out_type=jax.ShapeDtypeStruct((p_pad, C), jnp.float32),
        scratch_types=[
            pltpu.VMEM((per_w,), jnp.float32) for _ in range(4)
        ] + [pltpu.VMEM((KCH,), jnp.int32) for _ in range(8)
        ] + [pltpu.VMEM((KCH, CP), jnp.float32) for _ in range(8)] + [
            pltpu.VMEM((KCH, C), jnp.float32),
            pltpu.VMEM((KCH, C), jnp.float32),
        ] + [pltpu.SemaphoreType.DMA for _ in range(6)],
    )
    def sc_kernel(F, i0, i1, i2, i3, w0, w1, w2, w3, out,
                  sw0, sw1, sw2, sw3,
                  x00, x01, x02, x03, x10, x11, x12, x13,
                  t00, t01, t02, t03, t10, t11, t12, t13,
                  ov0, ov1, sg0, sg1, so0, so1, ss0, ss1):
        wid = lax.axis_index("s") * NC + lax.axis_index("c")
        base = wid * per_w
        nch = per_w // KCH
        iref = (i0, i1, i2, i3)
        wref = (w0, w1, w2, w3)
        stgw = (sw0, sw1, sw2, sw3)
        sidx = ((x00, x01, x02, x03), (x10, x11, x12, x13))
        taps = ((t00, t01, t02, t03), (t10, t11, t12, t13))
        ov = (ov0, ov1)
        gsem = (sg0, sg1)
        osem = (so0, so1)
        ssem = (ss0, ss1)

        for t in range(4):
            pltpu.sync_copy(wref[t].at[pl.ds(base, per_w)], stgw[t])

        def stage(b, ci):
            return [
                pltpu.make_async_copy(
                    iref[t].at[pl.ds(base + ci * KCH, KCH)],
                    sidx[b][t], ssem[b])
                for t in range(4)
            ]

        def gathers(b, ci):
            del ci
            return [
                pltpu.make_async_copy(
                    F.at[sidx[b][t].at[pl.ds(0, KCH)]],
                    taps[b][t], gsem[b])
                for t in range(4)
            ]

        def store(b, ci):
            return pltpu.make_async_copy(
                ov[b], out.at[pl.ds(base + ci * KCH, KCH)], osem[b])

        def combine(b, ci):
            tb = taps[b]
            ovb = ov[b]
            wbase = ci * KCH

            def group_body(g, carry):
                gsl = pl.ds(wbase + g * LANES, LANES)
                gw = [stgw[t][gsl] for t in range(4)]
                for j in range(LANES):
                    p = g * LANES + j
                    pv = jnp.full((LANES,), j, jnp.int32)
                    a = [_lane_bcast(gw[t], pv) for t in range(4)]
                    for cb in range(C // LANES):
                        s16 = pl.ds(cb * LANES, LANES)
                        acc = None
                        for t in range(4):
                            u = tb[t][p, s16]
                            acc = a[t] * u if acc is None else acc + a[t] * u
                        ovb[p, s16] = acc
                return carry

            lax.fori_loop(0, KCH // LANES, group_body, 0, unroll=False)

        for cp in stage(0, 0):
            cp.start()
        for cp in stage(0, 0):
            cp.wait()
        for cp in gathers(0, 0):
            cp.start()
        for cp in stage(1, 1):
            cp.start()

        def half(ci, b):
            nb = 1 - b
            for cp in gathers(b, ci):
                cp.wait()

            @pl.when(ci + 1 < nch)
            def _():
                for cp in stage(nb, ci + 1):
                    cp.wait()
                for cp in gathers(nb, ci + 1):
                    cp.start()

            @pl.when(ci + 2 < nch)
            def _():
                for cp in stage(b, ci + 2):
                    cp.start()

            @pl.when(ci >= 2)
            def _():
                store(b, ci - 2).wait()

            combine(b, ci)
            store(b, ci).start()

        def body(ci2, carry):
            half(ci2 * 2, 0)
            half(ci2 * 2 + 1, 1)
            return carry

        lax.fori_loop(0, nch // 2, body, 0, unroll=False)
        store(0, nch - 2).wait()
        store(1, nch - 1).wait()

    return sc_kernel


def kernel(image_shape, boxes, scores, fpn0, fpn1, fpn2, fpn3, fpn4):
    del scores
    fpns = (fpn0, fpn1, fpn2, fpn3, fpn4)
    B, N = boxes.shape[0], boxes.shape[1]
    C = fpn0.shape[-1]
    img = image_shape.astype(jnp.float32)
    img_h, img_w = img[1], img[2]

    npad = BN
    while npad < N or (npad * PTS) % (NW * KCH) != 0:
        npad += BN
    p_pad = npad * PTS

    CP = ((C + 127) // 128) * 128
    p_real = N * PTS
    sc_fn = _make_sc_kernel(p_real, p_pad, C, CP)

    outs = []
    for b in range(B):
        F = jnp.concatenate([f[b].reshape(-1, C) for f in fpns], axis=0)
        if CP != C:
            F = jnp.pad(F, ((0, 0), (0, CP - C)))
        levels, iy, ix = _level_coords(boxes[b], img_h, img_w)
        iy_e = jnp.pad(jnp.repeat(iy, CROP, axis=1),
                       ((0, npad - N), (0, 0)), constant_values=-1.0)
        ix_e = jnp.pad(jnp.tile(ix, (1, CROP)),
                       ((0, npad - N), (0, 0)), constant_values=-1.0)
        lvl2 = jnp.pad(levels[:, None], ((0, npad - N), (0, 0)))
        prep = _tc_prep(iy_e, ix_e, lvl2, npad)
        flat = [o.reshape(npad * PTS) for o in prep]
        out_flat = sc_fn(F, *flat)
        outs.append(out_flat[:N * PTS].reshape(N, CROP, CROP, C))
    return jnp.stack(outs, axis=0)
